# baseline (speedup 1.0000x reference)
import jax
import jax.numpy as jnp
from jax import lax
from jax.experimental import pallas as pl
from jax.experimental.pallas import tpu as pltpu
from jax.experimental.pallas import tpu_sc as plsc

BATCH = 16384
FACTOR = 32
ENVS = 4
WBCOL = 48
L = 16
NW = 32
B_PER_W = BATCH // NW
CHUNK = 128
N_CHUNKS = B_PER_W // CHUNK
GROUPS = B_PER_W // L

_LN2 = 0.6931471805599453


def _log_pos(x):
    bits = lax.bitcast_convert_type(x, jnp.int32)
    e = lax.shift_right_arithmetic(bits, 23) - 127
    mbits = lax.bitwise_or(lax.bitwise_and(bits, 0x007FFFFF), 0x3F800000)
    m = lax.bitcast_convert_type(mbits, jnp.float32)
    z = (m - 1.0) / (m + 1.0)
    z2 = z * z
    logm = 2.0 * z * (1.0 + z2 * (1.0 / 3.0 + z2 * (0.2 + z2 * (1.0 / 7.0))))
    return e.astype(jnp.float32) * _LN2 + logm


def _sigmoid(x):
    return 1.0 / (1.0 + jnp.exp(-x))


def _body(u_inv_hbm, i_inv_hbm, u_env_hbm, i_env_hbm, env_hbm, wb_hbm,
          uid_hbm, iid_hbm, eid_hbm,
          out_inv, out_env, out_cls,
          uid_v, iid_v, eid_v,
          ui_rows, ii_rows, ue_rows, ie_rows,
          env_v, wb_v, o1_v, o2_v, o3_v, sem):
    nc = lax.axis_size("c")
    wid = lax.axis_index("s") * nc + lax.axis_index("c")
    base = wid * B_PER_W

    for c in range(N_CHUNKS):
        pltpu.sync_copy(uid_hbm.at[pl.ds(base + c * CHUNK, CHUNK)], uid_v.at[c])
        pltpu.sync_copy(iid_hbm.at[pl.ds(base + c * CHUNK, CHUNK)], iid_v.at[c])
    pltpu.sync_copy(eid_hbm.at[pl.ds(base, B_PER_W)], eid_v)
    pltpu.sync_copy(env_hbm, env_v)
    pltpu.sync_copy(wb_hbm, wb_v)

    descs = []
    for c in range(N_CHUNKS):
        sl = pl.ds(c * CHUNK, CHUNK)
        for tbl, idx, rows in ((u_inv_hbm, uid_v, ui_rows),
                               (i_inv_hbm, iid_v, ii_rows),
                               (u_env_hbm, uid_v, ue_rows),
                               (i_env_hbm, iid_v, ie_rows)):
            descs.append(pltpu.async_copy(tbl.at[idx.at[c]], rows.at[sl], sem))
    for d in descs:
        d.wait()

    lanes = lax.iota(jnp.int32, L)

    w_lo = [wb_v[pl.ds(e * WBCOL, L)] for e in range(ENVS)]
    w_hi = [wb_v[pl.ds(e * WBCOL + L, L)] for e in range(ENVS)]
    w_b = [wb_v[pl.ds(e * WBCOL + 2 * L, L)] for e in range(ENVS)]

    def group(g, carry):
        bvec = g * L + lanes
        evec = eid_v[pl.ds(g * L, L)]

        zero = jnp.zeros((L,), jnp.float32)
        inv_s = zero
        env_s = zero
        logit = [zero, zero, zero, zero]
        for j in range(FACTOR):
            jv = jnp.full((L,), j, jnp.int32)
            u = plsc.load_gather(ui_rows, [bvec, jv])
            it = plsc.load_gather(ii_rows, [bvec, jv])
            p = u * it
            inv_s = inv_s + p
            wrow = w_lo if j < L else w_hi
            for e in range(ENVS):
                logit[e] = logit[e] + p * wrow[e][j % L]
            ue = plsc.load_gather(ue_rows, [bvec, jv])
            ie = plsc.load_gather(ie_rows, [bvec, jv])
            ev = plsc.load_gather(env_v, [evec, jv])
            env_s = env_s + ue * ie * ev

        inv_score = _sigmoid(inv_s)
        env_score = inv_score * _sigmoid(env_s)
        o1_v[pl.ds(g * L, L)] = inv_score
        o2_v[pl.ds(g * L, L)] = env_score

        for e in range(ENVS):
            logit[e] = logit[e] + w_b[e][0]
        m = jnp.maximum(jnp.maximum(logit[0], logit[1]),
                        jnp.maximum(logit[2], logit[3]))
        s = zero
        for e in range(ENVS):
            s = s + jnp.exp(logit[e] - m)
        lse = m + _log_pos(s)
        for e in range(ENVS):
            plsc.store_scatter(o3_v, [bvec, jnp.full((L,), e, jnp.int32)],
                               logit[e] - lse)
        return carry

    lax.fori_loop(0, GROUPS, group, 0)

    pltpu.sync_copy(o1_v, out_inv.at[pl.ds(base, B_PER_W)])
    pltpu.sync_copy(o2_v, out_env.at[pl.ds(base, B_PER_W)])
    pltpu.sync_copy(o3_v, out_cls.at[pl.ds(base, B_PER_W)])


def kernel(embed_user_invariant, embed_item_invariant, embed_user_env_aware,
           embed_item_env_aware, embed_env, cls_w, cls_b,
           users_id, items_id, envs_id, alpha):
    del alpha
    wb = jnp.concatenate(
        [cls_w, cls_b[:, None], jnp.zeros((ENVS, WBCOL - FACTOR - 1),
                                          jnp.float32)], axis=1).reshape(-1)

    mesh = plsc.VectorSubcoreMesh(core_axis_name="c", subcore_axis_name="s")
    run = pl.kernel(
        _body,
        out_type=(
            jax.ShapeDtypeStruct((BATCH,), jnp.float32),
            jax.ShapeDtypeStruct((BATCH,), jnp.float32),
            jax.ShapeDtypeStruct((BATCH, ENVS), jnp.float32),
        ),
        mesh=mesh,
        compiler_params=pltpu.CompilerParams(use_tc_tiling_on_sc=False,
                                             needs_layout_passes=False),
        scratch_types=[
            pltpu.VMEM((N_CHUNKS, CHUNK), jnp.int32),
            pltpu.VMEM((N_CHUNKS, CHUNK), jnp.int32),
            pltpu.VMEM((B_PER_W,), jnp.int32),
            pltpu.VMEM((B_PER_W, FACTOR), jnp.float32),
            pltpu.VMEM((B_PER_W, FACTOR), jnp.float32),
            pltpu.VMEM((B_PER_W, FACTOR), jnp.float32),
            pltpu.VMEM((B_PER_W, FACTOR), jnp.float32),
            pltpu.VMEM((ENVS, FACTOR), jnp.float32),
            pltpu.VMEM((ENVS * WBCOL,), jnp.float32),
            pltpu.VMEM((B_PER_W,), jnp.float32),
            pltpu.VMEM((B_PER_W,), jnp.float32),
            pltpu.VMEM((B_PER_W, ENVS), jnp.float32),
            pltpu.SemaphoreType.DMA,
        ],
    )
    return tuple(run(embed_user_invariant, embed_item_invariant,
                     embed_user_env_aware, embed_item_env_aware,
                     embed_env, wb, users_id, items_id, envs_id))
